# Initial kernel scaffold; baseline (speedup 1.0000x reference)
import jax
import jax.numpy as jnp
from jax.experimental import pallas as pl


def kernel(x, edge_index, W1, b1, W2, b2, W3, b3):
    raise NotImplementedError("write your pallas kernel here")


# speedup vs baseline: 12.6416x; 12.6416x over previous
import functools

import jax
import jax.numpy as jnp
from jax import lax
from jax.experimental import pallas as pl
from jax.experimental.pallas import tpu as pltpu
from jax.experimental.pallas import tpu_sc as plsc

N = 10000
E = 320000
F_IN = 128
HID = 64
C = 64

NC = 2
NS = 16
NW = NC * NS
EPT = E // NW
K = 80
NCH = EPT // K
NSTR = 10
SPL = N // NSTR

_mesh = plsc.VectorSubcoreMesh(core_axis_name="c", subcore_axis_name="s")


def _deg_body(dst_hbm, zeros_hbm, deg_out, dst_v, ones_v, acc_sh):
    c = lax.axis_index("c")
    s = lax.axis_index("s")
    wid = c * NS + s

    @pl.when(s == 0)
    def _zero():
        pltpu.sync_copy(zeros_hbm, acc_sh)

    for j in range(K // 16):
        ones_v[pl.ds(16 * j, 16)] = jnp.ones((16,), jnp.float32)

    plsc.subcore_barrier()

    @pl.loop(0, NCH)
    def _chunk(i):
        base = pl.multiple_of(wid * EPT + i * K, 8)
        pltpu.sync_copy(dst_hbm.at[pl.ds(base, K)], dst_v)
        pltpu.sync_copy(ones_v, acc_sh.at[dst_v], add=True)

    plsc.subcore_barrier()

    @pl.when(s == 0)
    def _out():
        pltpu.sync_copy(acc_sh, deg_out.at[c])


_deg_call = pl.kernel(
    _deg_body,
    out_type=jax.ShapeDtypeStruct((NC, N), jnp.float32),
    mesh=_mesh,
    scratch_types=[
        pltpu.VMEM((K,), jnp.int32),
        pltpu.VMEM((K,), jnp.float32),
        pltpu.VMEM_SHARED((N,), jnp.float32),
    ],
)


def _agg_body(hs_hbm, src_hbm, dst_hbm, zeros_hbm, out_hbm,
              src_v, dst_v, rows_v, sem, acc_sh):
    c = lax.axis_index("c")
    s = lax.axis_index("s")
    wid = c * NS + s

    @pl.when(s < NSTR)
    def _zero():
        pltpu.sync_copy(zeros_hbm.at[pl.ds(s * SPL, SPL)],
                        acc_sh.at[pl.ds(s * SPL, SPL)])

    plsc.subcore_barrier()

    @pl.loop(0, NCH)
    def _chunk(i):
        base = pl.multiple_of(wid * EPT + i * K, 8)
        pltpu.sync_copy(src_hbm.at[pl.ds(base, K)], src_v)
        pltpu.sync_copy(dst_hbm.at[pl.ds(base, K)], dst_v)
        pltpu.async_copy(hs_hbm.at[src_v], rows_v, sem).wait()
        pltpu.sync_copy(rows_v, acc_sh.at[dst_v], add=True)

    plsc.subcore_barrier()

    @pl.when(s < NSTR)
    def _out():
        pltpu.sync_copy(acc_sh.at[pl.ds(s * SPL, SPL)],
                        out_hbm.at[c, pl.ds(s * SPL, SPL)])


_agg_call = pl.kernel(
    _agg_body,
    out_type=jax.ShapeDtypeStruct((NC, N, HID), jnp.float32),
    mesh=_mesh,
    compiler_params=pltpu.CompilerParams(use_tc_tiling_on_sc=False),
    scratch_types=[
        pltpu.VMEM((K,), jnp.int32),
        pltpu.VMEM((K,), jnp.int32),
        pltpu.VMEM((K, HID), jnp.float32),
        pltpu.SemaphoreType.DMA,
        pltpu.VMEM_SHARED((N, HID), jnp.float32),
    ],
)


RB = 2000


def _mm1_body(d0_ref, d1_ref, x_ref, w_ref, hs_ref, dinv_ref):
    dinv = lax.rsqrt(d0_ref[...] + d1_ref[...] + 1.0)
    h = jnp.dot(x_ref[...], w_ref[...], preferred_element_type=jnp.float32)
    hs_ref[...] = h * dinv
    dinv_ref[...] = dinv


def _mid_body(a0_ref, a1_ref, hs_ref, dinv_ref, b_ref, w_ref, out_ref):
    dinv = dinv_ref[...]
    z = dinv * (a0_ref[...] + a1_ref[...] + hs_ref[...]) + b_ref[...]
    a = jnp.maximum(z, 0.0)
    out_ref[...] = dinv * jnp.dot(a, w_ref[...],
                                  preferred_element_type=jnp.float32)


def _fin_body(a0_ref, a1_ref, hs_ref, dinv_ref, b_ref, out_ref):
    z = dinv_ref[...] * (a0_ref[...] + a1_ref[...] + hs_ref[...]) + b_ref[...]
    m = jnp.max(z, axis=1, keepdims=True)
    e = jnp.exp(z - m)
    lse = jnp.log(jnp.sum(e, axis=1, keepdims=True)) + m
    out_ref[...] = z - lse


def _row_spec(cols):
    return pl.BlockSpec((RB, cols), lambda i: (i, 0))


def _full_spec(rows, cols):
    return pl.BlockSpec((rows, cols), lambda i: (0, 0))


_mm1 = pl.pallas_call(
    _mm1_body,
    grid=(N // RB,),
    in_specs=[_row_spec(1), _row_spec(1), _row_spec(F_IN),
              _full_spec(F_IN, HID)],
    out_specs=[_row_spec(HID), _row_spec(1)],
    out_shape=[jax.ShapeDtypeStruct((N, HID), jnp.float32),
               jax.ShapeDtypeStruct((N, 1), jnp.float32)],
)

_mid = pl.pallas_call(
    _mid_body,
    grid=(N // RB,),
    in_specs=[_row_spec(HID), _row_spec(HID), _row_spec(HID), _row_spec(1),
              _full_spec(1, HID), _full_spec(HID, HID)],
    out_specs=_row_spec(HID),
    out_shape=jax.ShapeDtypeStruct((N, HID), jnp.float32),
)

_fin = pl.pallas_call(
    _fin_body,
    grid=(N // RB,),
    in_specs=[_row_spec(C), _row_spec(C), _row_spec(C), _row_spec(1),
              _full_spec(1, C)],
    out_specs=_row_spec(C),
    out_shape=jax.ShapeDtypeStruct((N, C), jnp.float32),
)


def kernel(x, edge_index, W1, b1, W2, b2, W3, b3):
    src = edge_index[0]
    dst = edge_index[1]
    zeros_n = jnp.zeros((N,), jnp.float32)
    zeros_nh = jnp.zeros((N, HID), jnp.float32)

    deg_pair = _deg_call(dst, zeros_n)
    d0 = deg_pair[0].reshape(N, 1)
    d1 = deg_pair[1].reshape(N, 1)

    hs1, dinv = _mm1(d0, d1, x, W1)
    agg1 = _agg_call(hs1, src, dst, zeros_nh)
    hs2 = _mid(agg1[0], agg1[1], hs1, dinv, b1.reshape(1, HID), W2)
    agg2 = _agg_call(hs2, src, dst, zeros_nh)
    hs3 = _mid(agg2[0], agg2[1], hs2, dinv, b2.reshape(1, HID), W3)
    agg3 = _agg_call(hs3, src, dst, zeros_nh)
    return _fin(agg3[0], agg3[1], hs3, dinv, b3.reshape(1, C))


# speedup vs baseline: 35.6917x; 2.8234x over previous
import functools

import jax
import jax.numpy as jnp
from jax import lax
from jax.experimental import pallas as pl
from jax.experimental.pallas import tpu as pltpu
from jax.experimental.pallas import tpu_sc as plsc

N = 10000
E = 320000
F_IN = 128
HID = 64
C = 64

NC = 2
NS = 16
NW = NC * NS
EPT = E // NW
K = 80
NCH = EPT // K
NSTR = 10
SPL = N // NSTR

_mesh = plsc.VectorSubcoreMesh(core_axis_name="c", subcore_axis_name="s")


def _deg_body(dst3_hbm, zeros_hbm, deg_out, dst_all, ones_v, sem, acc_sh):
    c = lax.axis_index("c")
    s = lax.axis_index("s")
    wid = c * NS + s

    @pl.when(s == 0)
    def _zero():
        pltpu.sync_copy(zeros_hbm, acc_sh)

    for j in range(K // 16):
        ones_v[pl.ds(16 * j, 16)] = jnp.ones((16,), jnp.float32)

    pltpu.sync_copy(dst3_hbm.at[wid], dst_all)
    plsc.subcore_barrier()

    @pl.loop(0, NCH)
    def _fire(i):
        pltpu.async_copy(ones_v, acc_sh.at[dst_all.at[i]], sem, add=True)

    @pl.loop(0, NCH)
    def _drain(i):
        pltpu.make_async_copy(ones_v, acc_sh.at[dst_all.at[0]], sem).wait()

    plsc.subcore_barrier()

    @pl.when(s == 0)
    def _out():
        pltpu.sync_copy(acc_sh, deg_out.at[c])


_deg_call = pl.kernel(
    _deg_body,
    out_type=jax.ShapeDtypeStruct((NC, N), jnp.float32),
    mesh=_mesh,
    scratch_types=[
        pltpu.VMEM((NCH, K), jnp.int32),
        pltpu.VMEM((K,), jnp.float32),
        pltpu.SemaphoreType.DMA,
        pltpu.VMEM_SHARED((N,), jnp.float32),
    ],
)


NBUF = 5
NRINGS = NCH // NBUF


def _agg_body(hs_hbm, src_hbm, dst3_hbm, zeros_hbm, out_hbm,
              src_all, dst_all, rows, gsem, ssem, acc_sh):
    c = lax.axis_index("c")
    s = lax.axis_index("s")
    wid = c * NS + s

    @pl.when(s < NSTR)
    def _zero():
        pltpu.sync_copy(zeros_hbm.at[pl.ds(s * SPL, SPL)],
                        acc_sh.at[pl.ds(s * SPL, SPL)])

    pltpu.sync_copy(src_hbm.at[pl.ds(wid * EPT, EPT)], src_all)
    pltpu.sync_copy(dst3_hbm.at[wid], dst_all)
    plsc.subcore_barrier()

    def gather_start(i, b):
        pltpu.async_copy(hs_hbm.at[src_all.at[pl.ds(i * K, K)]],
                         rows.at[b], gsem.at[b])

    def gather_wait(b):
        pltpu.make_async_copy(hs_hbm.at[src_all.at[pl.ds(0, K)]],
                              rows.at[b], gsem.at[b]).wait()

    def scatter_start(i, b):
        pltpu.async_copy(rows.at[b], acc_sh.at[dst_all.at[i]],
                         ssem.at[b], add=True)

    def scatter_wait(b):
        pltpu.make_async_copy(rows.at[b], acc_sh.at[dst_all.at[0]],
                              ssem.at[b]).wait()

    for b in range(NBUF):
        gather_start(b, b)

    @pl.loop(0, NRINGS)
    def _ring(g):
        i0 = g * NBUF
        for b in range(NBUF):
            gather_wait(b)
            scatter_start(i0 + b, b)
        for b in range(NBUF):
            nxt = i0 + NBUF + b

            @pl.when(nxt < NCH)
            def _prefetch(nxt=nxt, b=b):
                scatter_wait(b)
                gather_start(nxt, b)

    for b in range(NBUF):
        scatter_wait(b)

    plsc.subcore_barrier()

    @pl.when(s < NSTR)
    def _out():
        pltpu.sync_copy(acc_sh.at[pl.ds(s * SPL, SPL)],
                        out_hbm.at[c, pl.ds(s * SPL, SPL)])


_agg_call = pl.kernel(
    _agg_body,
    out_type=jax.ShapeDtypeStruct((NC, N, HID), jnp.float32),
    mesh=_mesh,
    compiler_params=pltpu.CompilerParams(use_tc_tiling_on_sc=False),
    scratch_types=[
        pltpu.VMEM((EPT,), jnp.int32),
        pltpu.VMEM((NCH, K), jnp.int32),
        pltpu.VMEM((NBUF, K, HID), jnp.float32),
        pltpu.SemaphoreType.DMA((NBUF,)),
        pltpu.SemaphoreType.DMA((NBUF,)),
        pltpu.VMEM_SHARED((N, HID), jnp.float32),
    ],
)


RB = 2000


def _mm1_body(d0_ref, d1_ref, x_ref, w_ref, hs_ref, dinv_ref):
    dinv = lax.rsqrt(d0_ref[...] + d1_ref[...] + 1.0)
    h = jnp.dot(x_ref[...], w_ref[...], preferred_element_type=jnp.float32)
    hs_ref[...] = h * dinv
    dinv_ref[...] = dinv


def _mid_body(a0_ref, a1_ref, hs_ref, dinv_ref, b_ref, w_ref, out_ref):
    dinv = dinv_ref[...]
    z = dinv * (a0_ref[...] + a1_ref[...] + hs_ref[...]) + b_ref[...]
    a = jnp.maximum(z, 0.0)
    out_ref[...] = dinv * jnp.dot(a, w_ref[...],
                                  preferred_element_type=jnp.float32)


def _fin_body(a0_ref, a1_ref, hs_ref, dinv_ref, b_ref, out_ref):
    z = dinv_ref[...] * (a0_ref[...] + a1_ref[...] + hs_ref[...]) + b_ref[...]
    m = jnp.max(z, axis=1, keepdims=True)
    e = jnp.exp(z - m)
    lse = jnp.log(jnp.sum(e, axis=1, keepdims=True)) + m
    out_ref[...] = z - lse


def _row_spec(cols):
    return pl.BlockSpec((RB, cols), lambda i: (i, 0))


def _full_spec(rows, cols):
    return pl.BlockSpec((rows, cols), lambda i: (0, 0))


_mm1 = pl.pallas_call(
    _mm1_body,
    grid=(N // RB,),
    in_specs=[_row_spec(1), _row_spec(1), _row_spec(F_IN),
              _full_spec(F_IN, HID)],
    out_specs=[_row_spec(HID), _row_spec(1)],
    out_shape=[jax.ShapeDtypeStruct((N, HID), jnp.float32),
               jax.ShapeDtypeStruct((N, 1), jnp.float32)],
)

_mid = pl.pallas_call(
    _mid_body,
    grid=(N // RB,),
    in_specs=[_row_spec(HID), _row_spec(HID), _row_spec(HID), _row_spec(1),
              _full_spec(1, HID), _full_spec(HID, HID)],
    out_specs=_row_spec(HID),
    out_shape=jax.ShapeDtypeStruct((N, HID), jnp.float32),
)

_fin = pl.pallas_call(
    _fin_body,
    grid=(N // RB,),
    in_specs=[_row_spec(C), _row_spec(C), _row_spec(C), _row_spec(1),
              _full_spec(1, C)],
    out_specs=_row_spec(C),
    out_shape=jax.ShapeDtypeStruct((N, C), jnp.float32),
)


def kernel(x, edge_index, W1, b1, W2, b2, W3, b3):
    src = edge_index[0]
    dst = edge_index[1]
    dst3 = dst.reshape(NW, NCH, K)
    zeros_n = jnp.zeros((N,), jnp.float32)
    zeros_nh = jnp.zeros((N, HID), jnp.float32)

    deg_pair = _deg_call(dst3, zeros_n)
    d0 = deg_pair[0].reshape(N, 1)
    d1 = deg_pair[1].reshape(N, 1)

    hs1, dinv = _mm1(d0, d1, x, W1)
    agg1 = _agg_call(hs1, src, dst3, zeros_nh)
    hs2 = _mid(agg1[0], agg1[1], hs1, dinv, b1.reshape(1, HID), W2)
    agg2 = _agg_call(hs2, src, dst3, zeros_nh)
    hs3 = _mid(agg2[0], agg2[1], hs2, dinv, b2.reshape(1, HID), W3)
    agg3 = _agg_call(hs3, src, dst3, zeros_nh)
    return _fin(agg3[0], agg3[1], hs3, dinv, b3.reshape(1, C))


# speedup vs baseline: 38.2099x; 1.0706x over previous
import functools

import jax
import jax.numpy as jnp
from jax import lax
from jax.experimental import pallas as pl
from jax.experimental.pallas import tpu as pltpu
from jax.experimental.pallas import tpu_sc as plsc

N = 10000
E = 320000
F_IN = 128
HID = 64
C = 64

NC = 2
NS = 16
NW = NC * NS
EPT = E // NW
KB = 128
NFULL = EPT // KB
TAIL = EPT - NFULL * KB
NBUF = 3
NRINGS = NFULL // NBUF
SP_BUFS = (0,)
TPL = N // NS
NSTR = 10
SPL = N // NSTR

_mesh = plsc.VectorSubcoreMesh(core_axis_name="c", subcore_axis_name="s")


def _deg_body(ei_hbm, zeros_hbm, deg_out, dst_all, ones_v, sem, acc_sh):
    c = lax.axis_index("c")
    s = lax.axis_index("s")
    wid = c * NS + s

    @pl.when(s == 0)
    def _zero():
        pltpu.sync_copy(zeros_hbm, acc_sh)

    for j in range(KB // 16):
        ones_v[pl.ds(16 * j, 16)] = jnp.ones((16,), jnp.float32)

    pltpu.sync_copy(ei_hbm.at[1, pl.ds(wid * EPT, EPT)], dst_all)
    plsc.subcore_barrier()

    @pl.loop(0, NFULL)
    def _fire(i):
        pltpu.async_copy(ones_v, acc_sh.at[dst_all.at[pl.ds(i * KB, KB)]],
                         sem, add=True)

    pltpu.async_copy(ones_v.at[pl.ds(0, TAIL)],
                     acc_sh.at[dst_all.at[pl.ds(NFULL * KB, TAIL)]],
                     sem, add=True)

    @pl.loop(0, NFULL)
    def _drain(i):
        pltpu.make_async_copy(ones_v, acc_sh.at[dst_all.at[pl.ds(0, KB)]],
                              sem).wait()

    pltpu.make_async_copy(ones_v.at[pl.ds(0, TAIL)],
                          acc_sh.at[dst_all.at[pl.ds(0, TAIL)]], sem).wait()

    plsc.subcore_barrier()

    @pl.when(s == 0)
    def _out():
        pltpu.sync_copy(acc_sh, deg_out.at[c])


_deg_call = pl.kernel(
    _deg_body,
    out_type=jax.ShapeDtypeStruct((NC, N), jnp.float32),
    mesh=_mesh,
    compiler_params=pltpu.CompilerParams(use_tc_tiling_on_sc=False),
    scratch_types=[
        pltpu.VMEM((EPT,), jnp.int32),
        pltpu.VMEM((KB,), jnp.float32),
        pltpu.SemaphoreType.DMA,
        pltpu.VMEM_SHARED((N,), jnp.float32),
    ],
)


def _agg_body(hs_hbm, ei_hbm, zeros_hbm, out_hbm,
              src_all, dst_all, rows, gsem, ssem, acc_sh, tab_sh):
    c = lax.axis_index("c")
    s = lax.axis_index("s")
    wid = c * NS + s

    @pl.when(s < NSTR)
    def _zero():
        pltpu.sync_copy(zeros_hbm.at[pl.ds(s * SPL, SPL)],
                        acc_sh.at[pl.ds(s * SPL, SPL)])

    pltpu.sync_copy(hs_hbm.at[pl.ds(s * TPL, TPL)],
                    tab_sh.at[pl.ds(s * TPL, TPL)])
    pltpu.sync_copy(ei_hbm.at[0, pl.ds(wid * EPT, EPT)], src_all)
    pltpu.sync_copy(ei_hbm.at[1, pl.ds(wid * EPT, EPT)], dst_all)
    plsc.subcore_barrier()

    def gather_start(i, b):
        tab = tab_sh if b in SP_BUFS else hs_hbm
        pltpu.async_copy(tab.at[src_all.at[pl.ds(i * KB, KB)]],
                         rows.at[b], gsem.at[b])

    def gather_wait(b):
        tab = tab_sh if b in SP_BUFS else hs_hbm
        pltpu.make_async_copy(tab.at[src_all.at[pl.ds(0, KB)]],
                              rows.at[b], gsem.at[b]).wait()

    def scatter_start(i, b):
        pltpu.async_copy(rows.at[b], acc_sh.at[dst_all.at[pl.ds(i * KB, KB)]],
                         ssem.at[b], add=True)

    def scatter_wait(b):
        pltpu.make_async_copy(rows.at[b], acc_sh.at[dst_all.at[pl.ds(0, KB)]],
                              ssem.at[b]).wait()

    for b in range(NBUF):
        gather_start(b, b)

    @pl.loop(0, NRINGS)
    def _ring(g):
        i0 = g * NBUF
        for b in range(NBUF):
            gather_wait(b)
            scatter_start(i0 + b, b)
        for b in range(NBUF):
            nxt = i0 + NBUF + b

            @pl.when(nxt < NFULL)
            def _prefetch(nxt=nxt, b=b):
                scatter_wait(b)
                gather_start(nxt, b)

    for b in range(NBUF):
        scatter_wait(b)

    pltpu.async_copy(hs_hbm.at[src_all.at[pl.ds(NFULL * KB, TAIL)]],
                     rows.at[0, pl.ds(0, TAIL)], gsem.at[0])
    pltpu.make_async_copy(hs_hbm.at[src_all.at[pl.ds(0, TAIL)]],
                          rows.at[0, pl.ds(0, TAIL)], gsem.at[0]).wait()
    pltpu.async_copy(rows.at[0, pl.ds(0, TAIL)],
                     acc_sh.at[dst_all.at[pl.ds(NFULL * KB, TAIL)]],
                     ssem.at[0], add=True)
    pltpu.make_async_copy(rows.at[0, pl.ds(0, TAIL)],
                          acc_sh.at[dst_all.at[pl.ds(0, TAIL)]],
                          ssem.at[0]).wait()

    plsc.subcore_barrier()

    @pl.when(s < NSTR)
    def _out():
        pltpu.sync_copy(acc_sh.at[pl.ds(s * SPL, SPL)],
                        out_hbm.at[c, pl.ds(s * SPL, SPL)])


_agg_call = pl.kernel(
    _agg_body,
    out_type=jax.ShapeDtypeStruct((NC, N, HID), jnp.float32),
    mesh=_mesh,
    compiler_params=pltpu.CompilerParams(use_tc_tiling_on_sc=False),
    scratch_types=[
        pltpu.VMEM((EPT,), jnp.int32),
        pltpu.VMEM((EPT,), jnp.int32),
        pltpu.VMEM((NBUF, KB, HID), jnp.float32),
        pltpu.SemaphoreType.DMA((NBUF,)),
        pltpu.SemaphoreType.DMA((NBUF,)),
        pltpu.VMEM_SHARED((N, HID), jnp.float32),
        pltpu.VMEM_SHARED((N, HID), jnp.float32),
    ],
)


NP = N // 2
PB = 1000
PW = 2 * HID


def _mm1_body(xp_ref, w_ref, dinv_ref, hs_ref):
    h = jnp.dot(xp_ref[...], w_ref[...], preferred_element_type=jnp.float32)
    hs_ref[...] = h * dinv_ref[...]


def _mid_body(a0_ref, a1_ref, hs_ref, dinv_ref, b_ref, w_ref, out_ref):
    dinv = dinv_ref[...]
    z = dinv * (a0_ref[...] + a1_ref[...] + hs_ref[...]) + b_ref[...]
    a = jnp.maximum(z, 0.0)
    out_ref[...] = dinv * jnp.dot(a, w_ref[...],
                                  preferred_element_type=jnp.float32)


def _fin_body(a0_ref, a1_ref, hs_ref, dinv_ref, b_ref, out_ref):
    z = dinv_ref[...] * (a0_ref[...] + a1_ref[...] + hs_ref[...]) + b_ref[...]
    z1 = z[:, :HID]
    z2 = z[:, HID:]

    def lsm(zz):
        m = jnp.max(zz, axis=1, keepdims=True)
        e = jnp.exp(zz - m)
        return zz - (jnp.log(jnp.sum(e, axis=1, keepdims=True)) + m)

    out_ref[...] = jnp.concatenate([lsm(z1), lsm(z2)], axis=1)


def _pspec(cols=PW):
    return pl.BlockSpec((PB, cols), lambda i: (i, 0))


def _wspec(rows, cols):
    return pl.BlockSpec((rows, cols), lambda i: (0, 0))


_a1spec = pl.BlockSpec((PB, PW), lambda i: (i + NP // PB, 0))

_mm1 = pl.pallas_call(
    _mm1_body,
    grid=(NP // PB,),
    in_specs=[_pspec(2 * F_IN), _wspec(2 * F_IN, PW), _pspec()],
    out_specs=_pspec(),
    out_shape=jax.ShapeDtypeStruct((NP, PW), jnp.float32),
)

_mid = pl.pallas_call(
    _mid_body,
    grid=(NP // PB,),
    in_specs=[_pspec(), _a1spec, _pspec(), _pspec(),
              _wspec(1, PW), _wspec(PW, PW)],
    out_specs=_pspec(),
    out_shape=jax.ShapeDtypeStruct((NP, PW), jnp.float32),
)

_fin = pl.pallas_call(
    _fin_body,
    grid=(NP // PB,),
    in_specs=[_pspec(), _a1spec, _pspec(), _pspec(), _wspec(1, PW)],
    out_specs=_pspec(),
    out_shape=jax.ShapeDtypeStruct((NP, PW), jnp.float32),
)


def _blockdiag(w):
    z = jnp.zeros_like(w)
    return jnp.concatenate(
        [jnp.concatenate([w, z], axis=1), jnp.concatenate([z, w], axis=1)],
        axis=0)


def kernel(x, edge_index, W1, b1, W2, b2, W3, b3):
    zeros_n = jnp.zeros((N,), jnp.float32)
    zeros_nh = jnp.zeros((N, HID), jnp.float32)

    xp = x.reshape(NP, 2 * F_IN)
    W1b = _blockdiag(W1)
    W2b = _blockdiag(W2)
    W3b = _blockdiag(W3)
    b1p = jnp.concatenate([b1, b1]).reshape(1, PW)
    b2p = jnp.concatenate([b2, b2]).reshape(1, PW)
    b3p = jnp.concatenate([b3, b3]).reshape(1, PW)

    deg_pair = _deg_call(edge_index, zeros_n)
    dinv = lax.rsqrt(deg_pair[0] + deg_pair[1] + 1.0)
    dinv_p = jnp.repeat(dinv, HID).reshape(NP, PW)

    hs1p = _mm1(xp, W1b, dinv_p)
    agg1 = _agg_call(hs1p.reshape(N, HID), edge_index, zeros_nh)
    aggv1 = agg1.reshape(N, PW)
    hs2p = _mid(aggv1, aggv1, hs1p, dinv_p, b1p, W2b)
    agg2 = _agg_call(hs2p.reshape(N, HID), edge_index, zeros_nh)
    aggv2 = agg2.reshape(N, PW)
    hs3p = _mid(aggv2, aggv2, hs2p, dinv_p, b2p, W3b)
    agg3 = _agg_call(hs3p.reshape(N, HID), edge_index, zeros_nh)
    aggv3 = agg3.reshape(N, PW)
    outp = _fin(aggv3, aggv3, hs3p, dinv_p, b3p)
    return outp.reshape(N, C)


# speedup vs baseline: 48.8236x; 1.2778x over previous
import functools

import jax
import jax.numpy as jnp
from jax import lax
from jax.experimental import pallas as pl
from jax.experimental.pallas import tpu as pltpu
from jax.experimental.pallas import tpu_sc as plsc

N = 10000
E = 320000
F_IN = 128
HID = 64
C = 64

NC = 2
NS = 16
NW = NC * NS
EPT = E // NW
KB = 128
NFULL = EPT // KB
TAIL = EPT - NFULL * KB
NBUF = 6
NRINGS = NFULL // NBUF
NSTR = 10
SPL = N // NSTR

_mesh = plsc.VectorSubcoreMesh(core_axis_name="c", subcore_axis_name="s")


def _deg_body(ei_hbm, zeros_hbm, deg_out, dst_all, ones_v, sem, acc_sh):
    c = lax.axis_index("c")
    s = lax.axis_index("s")
    wid = c * NS + s

    @pl.when(s == 0)
    def _zero():
        pltpu.sync_copy(zeros_hbm, acc_sh)

    for j in range(KB // 16):
        ones_v[pl.ds(16 * j, 16)] = jnp.ones((16,), jnp.float32)

    pltpu.sync_copy(ei_hbm.at[1, pl.ds(wid * EPT, EPT)], dst_all)
    plsc.subcore_barrier()

    @pl.loop(0, NFULL)
    def _fire(i):
        pltpu.async_copy(ones_v, acc_sh.at[dst_all.at[pl.ds(i * KB, KB)]],
                         sem, add=True)

    pltpu.async_copy(ones_v.at[pl.ds(0, TAIL)],
                     acc_sh.at[dst_all.at[pl.ds(NFULL * KB, TAIL)]],
                     sem, add=True)

    @pl.loop(0, NFULL)
    def _drain(i):
        pltpu.make_async_copy(ones_v, acc_sh.at[dst_all.at[pl.ds(0, KB)]],
                              sem).wait()

    pltpu.make_async_copy(ones_v.at[pl.ds(0, TAIL)],
                          acc_sh.at[dst_all.at[pl.ds(0, TAIL)]], sem).wait()

    plsc.subcore_barrier()

    @pl.when(s == 0)
    def _out():
        pltpu.sync_copy(acc_sh, deg_out.at[c])


_deg_call = pl.kernel(
    _deg_body,
    out_type=jax.ShapeDtypeStruct((NC, N), jnp.float32),
    mesh=_mesh,
    compiler_params=pltpu.CompilerParams(use_tc_tiling_on_sc=False),
    scratch_types=[
        pltpu.VMEM((EPT,), jnp.int32),
        pltpu.VMEM((KB,), jnp.float32),
        pltpu.SemaphoreType.DMA,
        pltpu.VMEM_SHARED((N,), jnp.float32),
    ],
)


def _agg_body(hs_hbm, ei_hbm, zeros_hbm, out_hbm,
              src_all, dst_all, rows, gsem, ssem, acc_sh):
    c = lax.axis_index("c")
    s = lax.axis_index("s")
    wid = c * NS + s

    @pl.when(s < NSTR)
    def _zero():
        pltpu.sync_copy(zeros_hbm.at[pl.ds(s * SPL, SPL)],
                        acc_sh.at[pl.ds(s * SPL, SPL)])

    pltpu.sync_copy(ei_hbm.at[0, pl.ds(wid * EPT, EPT)], src_all)
    pltpu.sync_copy(ei_hbm.at[1, pl.ds(wid * EPT, EPT)], dst_all)
    plsc.subcore_barrier()

    def gather_start(i, b):
        pltpu.async_copy(hs_hbm.at[src_all.at[pl.ds(i * KB, KB)]],
                         rows.at[b], gsem.at[b])

    def gather_wait(b):
        pltpu.make_async_copy(hs_hbm.at[src_all.at[pl.ds(0, KB)]],
                              rows.at[b], gsem.at[b]).wait()

    def scatter_start(i, b):
        pltpu.async_copy(rows.at[b], acc_sh.at[dst_all.at[pl.ds(i * KB, KB)]],
                         ssem.at[b], add=True)

    def scatter_wait(b):
        pltpu.make_async_copy(rows.at[b], acc_sh.at[dst_all.at[pl.ds(0, KB)]],
                              ssem.at[b]).wait()

    for b in range(NBUF):
        gather_start(b, b)

    @pl.loop(0, NRINGS)
    def _ring(g):
        i0 = g * NBUF
        for b in range(NBUF):
            gather_wait(b)
            scatter_start(i0 + b, b)
        for b in range(NBUF):
            nxt = i0 + NBUF + b

            @pl.when(nxt < NFULL)
            def _prefetch(nxt=nxt, b=b):
                scatter_wait(b)
                gather_start(nxt, b)

    for b in range(NBUF):
        scatter_wait(b)

    pltpu.async_copy(hs_hbm.at[src_all.at[pl.ds(NFULL * KB, TAIL)]],
                     rows.at[0, pl.ds(0, TAIL)], gsem.at[0])
    pltpu.make_async_copy(hs_hbm.at[src_all.at[pl.ds(0, TAIL)]],
                          rows.at[0, pl.ds(0, TAIL)], gsem.at[0]).wait()
    pltpu.async_copy(rows.at[0, pl.ds(0, TAIL)],
                     acc_sh.at[dst_all.at[pl.ds(NFULL * KB, TAIL)]],
                     ssem.at[0], add=True)
    pltpu.make_async_copy(rows.at[0, pl.ds(0, TAIL)],
                          acc_sh.at[dst_all.at[pl.ds(0, TAIL)]],
                          ssem.at[0]).wait()

    plsc.subcore_barrier()

    @pl.when(s < NSTR)
    def _out():
        pltpu.sync_copy(acc_sh.at[pl.ds(s * SPL, SPL)],
                        out_hbm.at[c, pl.ds(s * SPL, SPL)])


_agg_call = pl.kernel(
    _agg_body,
    out_type=jax.ShapeDtypeStruct((NC, N, HID), jnp.float32),
    mesh=_mesh,
    compiler_params=pltpu.CompilerParams(use_tc_tiling_on_sc=False),
    scratch_types=[
        pltpu.VMEM((EPT,), jnp.int32),
        pltpu.VMEM((EPT,), jnp.int32),
        pltpu.VMEM((NBUF, KB, HID), jnp.float32),
        pltpu.SemaphoreType.DMA((NBUF,)),
        pltpu.SemaphoreType.DMA((NBUF,)),
        pltpu.VMEM_SHARED((N, HID), jnp.float32),
    ],
)


NP = N // 2
PB = 1000
PW = 2 * HID


def _mm1_body(xp_ref, w_ref, dinv_ref, hs_ref):
    h = jnp.dot(xp_ref[...], w_ref[...], preferred_element_type=jnp.float32)
    hs_ref[...] = h * dinv_ref[...]


def _mid_body(a0_ref, a1_ref, hs_ref, dinv_ref, b_ref, w_ref, out_ref):
    dinv = dinv_ref[...]
    z = dinv * (a0_ref[...] + a1_ref[...] + hs_ref[...]) + b_ref[...]
    a = jnp.maximum(z, 0.0)
    out_ref[...] = dinv * jnp.dot(a, w_ref[...],
                                  preferred_element_type=jnp.float32)


def _fin_body(a0_ref, a1_ref, hs_ref, dinv_ref, b_ref, out_ref):
    z = dinv_ref[...] * (a0_ref[...] + a1_ref[...] + hs_ref[...]) + b_ref[...]
    z1 = z[:, :HID]
    z2 = z[:, HID:]

    def lsm(zz):
        m = jnp.max(zz, axis=1, keepdims=True)
        e = jnp.exp(zz - m)
        return zz - (jnp.log(jnp.sum(e, axis=1, keepdims=True)) + m)

    o1 = lsm(z1)
    o2 = lsm(z2)
    out_ref[...] = jnp.stack([o1, o2], axis=1).reshape(2 * PB, C)


def _pspec(cols=PW):
    return pl.BlockSpec((PB, cols), lambda i: (i, 0))


def _wspec(rows, cols):
    return pl.BlockSpec((rows, cols), lambda i: (0, 0))


_a1spec = pl.BlockSpec((PB, PW), lambda i: (i + NP // PB, 0))

_mm1 = pl.pallas_call(
    _mm1_body,
    grid=(NP // PB,),
    in_specs=[_pspec(2 * F_IN), _wspec(2 * F_IN, PW), _pspec()],
    out_specs=_pspec(),
    out_shape=jax.ShapeDtypeStruct((NP, PW), jnp.float32),
)

_mid = pl.pallas_call(
    _mid_body,
    grid=(NP // PB,),
    in_specs=[_pspec(), _a1spec, _pspec(), _pspec(),
              _wspec(1, PW), _wspec(PW, PW)],
    out_specs=_pspec(),
    out_shape=jax.ShapeDtypeStruct((NP, PW), jnp.float32),
)

_fin = pl.pallas_call(
    _fin_body,
    grid=(NP // PB,),
    in_specs=[_pspec(), _a1spec, _pspec(), _pspec(), _wspec(1, PW)],
    out_specs=pl.BlockSpec((2 * PB, C), lambda i: (i, 0)),
    out_shape=jax.ShapeDtypeStruct((N, C), jnp.float32),
)


def _blockdiag(w):
    z = jnp.zeros_like(w)
    return jnp.concatenate(
        [jnp.concatenate([w, z], axis=1), jnp.concatenate([z, w], axis=1)],
        axis=0)


def kernel(x, edge_index, W1, b1, W2, b2, W3, b3):
    zeros_n = jnp.zeros((N,), jnp.float32)
    zeros_nh = jnp.zeros((N, HID), jnp.float32)

    xp = x.reshape(NP, 2 * F_IN)
    W1b = _blockdiag(W1)
    W2b = _blockdiag(W2)
    W3b = _blockdiag(W3)
    b1p = jnp.concatenate([b1, b1]).reshape(1, PW)
    b2p = jnp.concatenate([b2, b2]).reshape(1, PW)
    b3p = jnp.concatenate([b3, b3]).reshape(1, PW)

    deg_pair = _deg_call(edge_index, zeros_n)
    dinv = lax.rsqrt(deg_pair[0] + deg_pair[1] + 1.0)
    dinv_p = jnp.repeat(dinv, HID).reshape(NP, PW)

    hs1p = _mm1(xp, W1b, dinv_p)
    agg1 = _agg_call(hs1p.reshape(N, HID), edge_index, zeros_nh)
    aggv1 = agg1.reshape(N, PW)
    hs2p = _mid(aggv1, aggv1, hs1p, dinv_p, b1p, W2b)
    agg2 = _agg_call(hs2p.reshape(N, HID), edge_index, zeros_nh)
    aggv2 = agg2.reshape(N, PW)
    hs3p = _mid(aggv2, aggv2, hs2p, dinv_p, b2p, W3b)
    agg3 = _agg_call(hs3p.reshape(N, HID), edge_index, zeros_nh)
    aggv3 = agg3.reshape(N, PW)
    return _fin(aggv3, aggv3, hs3p, dinv_p, b3p)


# speedup vs baseline: 49.1487x; 1.0067x over previous
import functools

import jax
import jax.numpy as jnp
from jax import lax
from jax.experimental import pallas as pl
from jax.experimental.pallas import tpu as pltpu
from jax.experimental.pallas import tpu_sc as plsc

N = 10000
E = 320000
F_IN = 128
HID = 64
C = 64

NC = 2
NS = 16
NW = NC * NS
EPT = E // NW
KB = 128
NFULL = EPT // KB
TAIL = EPT - NFULL * KB
NBUF = 6
NRINGS = NFULL // NBUF
NSTR = 10
SPL = N // NSTR

_mesh = plsc.VectorSubcoreMesh(core_axis_name="c", subcore_axis_name="s")


def _deg_body(ei_hbm, zeros_hbm, deg_out, dst_all, ones_v, sem, acc_sh):
    c = lax.axis_index("c")
    s = lax.axis_index("s")
    wid = c * NS + s

    @pl.when(s == 0)
    def _zero():
        pltpu.sync_copy(zeros_hbm, acc_sh)

    for j in range(KB // 16):
        ones_v[pl.ds(16 * j, 16)] = jnp.ones((16,), jnp.float32)

    pltpu.sync_copy(ei_hbm.at[1, pl.ds(wid * EPT, EPT)], dst_all)
    plsc.subcore_barrier()

    @pl.loop(0, NFULL)
    def _fire(i):
        pltpu.async_copy(ones_v, acc_sh.at[dst_all.at[pl.ds(i * KB, KB)]],
                         sem, add=True)

    pltpu.async_copy(ones_v.at[pl.ds(0, TAIL)],
                     acc_sh.at[dst_all.at[pl.ds(NFULL * KB, TAIL)]],
                     sem, add=True)

    @pl.loop(0, NFULL)
    def _drain(i):
        pltpu.make_async_copy(ones_v, acc_sh.at[dst_all.at[pl.ds(0, KB)]],
                              sem).wait()

    pltpu.make_async_copy(ones_v.at[pl.ds(0, TAIL)],
                          acc_sh.at[dst_all.at[pl.ds(0, TAIL)]], sem).wait()

    plsc.subcore_barrier()

    @pl.when(s == 0)
    def _out():
        pltpu.sync_copy(acc_sh, deg_out.at[c])


_deg_call = pl.kernel(
    _deg_body,
    out_type=jax.ShapeDtypeStruct((NC, N), jnp.float32),
    mesh=_mesh,
    compiler_params=pltpu.CompilerParams(use_tc_tiling_on_sc=False),
    scratch_types=[
        pltpu.VMEM((EPT,), jnp.int32),
        pltpu.VMEM((KB,), jnp.float32),
        pltpu.SemaphoreType.DMA,
        pltpu.VMEM_SHARED((N,), jnp.float32),
    ],
)


def _agg_body(hs_hbm, ei_hbm, zeros_hbm, out_hbm,
              src_all, dst_all, rows, gsem, ssem, acc_sh):
    c = lax.axis_index("c")
    s = lax.axis_index("s")
    wid = c * NS + s

    @pl.when(s < NSTR)
    def _zero():
        pltpu.sync_copy(zeros_hbm.at[pl.ds(s * SPL, SPL)],
                        acc_sh.at[pl.ds(s * SPL, SPL)])

    pltpu.sync_copy(ei_hbm.at[0, pl.ds(wid * EPT, EPT)], src_all)
    pltpu.sync_copy(ei_hbm.at[1, pl.ds(wid * EPT, EPT)], dst_all)
    plsc.subcore_barrier()

    def gather_start(i, b):
        pltpu.async_copy(hs_hbm.at[src_all.at[pl.ds(i * KB, KB)]],
                         rows.at[b], gsem.at[b])

    def gather_wait(b):
        pltpu.make_async_copy(hs_hbm.at[src_all.at[pl.ds(0, KB)]],
                              rows.at[b], gsem.at[b]).wait()

    def scatter_start(i, b):
        pltpu.async_copy(rows.at[b], acc_sh.at[dst_all.at[pl.ds(i * KB, KB)]],
                         ssem.at[b], add=True)

    def scatter_wait(b):
        pltpu.make_async_copy(rows.at[b], acc_sh.at[dst_all.at[pl.ds(0, KB)]],
                              ssem.at[b]).wait()

    for b in range(NBUF):
        gather_start(b, b)

    @pl.loop(0, NRINGS)
    def _ring(g):
        i0 = g * NBUF
        for b in range(NBUF):
            gather_wait(b)
            scatter_start(i0 + b, b)
        for b in range(NBUF):
            nxt = i0 + NBUF + b

            @pl.when(nxt < NFULL)
            def _prefetch(nxt=nxt, b=b):
                scatter_wait(b)
                gather_start(nxt, b)

    for b in range(NBUF):
        scatter_wait(b)

    pltpu.async_copy(hs_hbm.at[src_all.at[pl.ds(NFULL * KB, TAIL)]],
                     rows.at[0, pl.ds(0, TAIL)], gsem.at[0])
    pltpu.make_async_copy(hs_hbm.at[src_all.at[pl.ds(0, TAIL)]],
                          rows.at[0, pl.ds(0, TAIL)], gsem.at[0]).wait()
    pltpu.async_copy(rows.at[0, pl.ds(0, TAIL)],
                     acc_sh.at[dst_all.at[pl.ds(NFULL * KB, TAIL)]],
                     ssem.at[0], add=True)
    pltpu.make_async_copy(rows.at[0, pl.ds(0, TAIL)],
                          acc_sh.at[dst_all.at[pl.ds(0, TAIL)]],
                          ssem.at[0]).wait()

    plsc.subcore_barrier()

    @pl.when(s < NSTR)
    def _out():
        pltpu.sync_copy(acc_sh.at[pl.ds(s * SPL, SPL)],
                        out_hbm.at[c, pl.ds(s * SPL, SPL)])


_agg_call = pl.kernel(
    _agg_body,
    out_type=jax.ShapeDtypeStruct((NC, N, HID), jnp.float32),
    mesh=_mesh,
    compiler_params=pltpu.CompilerParams(use_tc_tiling_on_sc=False),
    scratch_types=[
        pltpu.VMEM((EPT,), jnp.int32),
        pltpu.VMEM((EPT,), jnp.int32),
        pltpu.VMEM((NBUF, KB, HID), jnp.float32),
        pltpu.SemaphoreType.DMA((NBUF,)),
        pltpu.SemaphoreType.DMA((NBUF,)),
        pltpu.VMEM_SHARED((N, HID), jnp.float32),
    ],
)


NP = N // 2
PB = 1000
PW = 2 * HID


def _mm1_body(xp_ref, w_ref, dinv_ref, hs_ref):
    h = jnp.dot(xp_ref[...], w_ref[...], preferred_element_type=jnp.float32)
    hs_ref[...] = h * dinv_ref[...]


def _mid_body(a0_ref, a1_ref, hs_ref, dinv_ref, b_ref, w_ref, out_ref):
    dinv = dinv_ref[...]
    z = dinv * (a0_ref[...] + a1_ref[...] + hs_ref[...]) + b_ref[...]
    a = jnp.maximum(z, 0.0)
    out_ref[...] = dinv * jnp.dot(a, w_ref[...],
                                  preferred_element_type=jnp.float32)


def _fin_body(a0_ref, a1_ref, hs_ref, dinv_ref, b_ref, out_ref):
    z = dinv_ref[...] * (a0_ref[...] + a1_ref[...] + hs_ref[...]) + b_ref[...]
    z1 = z[:, :HID]
    z2 = z[:, HID:]

    def lsm(zz):
        m = jnp.max(zz, axis=1, keepdims=True)
        e = jnp.exp(zz - m)
        return zz - (jnp.log(jnp.sum(e, axis=1, keepdims=True)) + m)

    out_ref[...] = jnp.concatenate([lsm(z1), lsm(z2)], axis=1)


def _pspec(cols=PW):
    return pl.BlockSpec((PB, cols), lambda i: (i, 0))


def _wspec(rows, cols):
    return pl.BlockSpec((rows, cols), lambda i: (0, 0))


_a1spec = pl.BlockSpec((PB, PW), lambda i: (i + NP // PB, 0))

_mm1 = pl.pallas_call(
    _mm1_body,
    grid=(NP // PB,),
    in_specs=[_pspec(2 * F_IN), _wspec(2 * F_IN, PW), _pspec()],
    out_specs=_pspec(),
    out_shape=jax.ShapeDtypeStruct((NP, PW), jnp.float32),
)

_mid = pl.pallas_call(
    _mid_body,
    grid=(NP // PB,),
    in_specs=[_pspec(), _a1spec, _pspec(), _pspec(),
              _wspec(1, PW), _wspec(PW, PW)],
    out_specs=_pspec(),
    out_shape=jax.ShapeDtypeStruct((NP, PW), jnp.float32),
)

_fin = pl.pallas_call(
    _fin_body,
    grid=(NP // PB,),
    in_specs=[_pspec(), _a1spec, _pspec(), _pspec(), _wspec(1, PW)],
    out_specs=_pspec(),
    out_shape=jax.ShapeDtypeStruct((NP, PW), jnp.float32),
)


def _blockdiag(w):
    z = jnp.zeros_like(w)
    return jnp.concatenate(
        [jnp.concatenate([w, z], axis=1), jnp.concatenate([z, w], axis=1)],
        axis=0)


def kernel(x, edge_index, W1, b1, W2, b2, W3, b3):
    zeros_n = jnp.zeros((N,), jnp.float32)
    zeros_nh = jnp.zeros((N, HID), jnp.float32)

    xp = x.reshape(NP, 2 * F_IN)
    W1b = _blockdiag(W1)
    W2b = _blockdiag(W2)
    W3b = _blockdiag(W3)
    b1p = jnp.concatenate([b1, b1]).reshape(1, PW)
    b2p = jnp.concatenate([b2, b2]).reshape(1, PW)
    b3p = jnp.concatenate([b3, b3]).reshape(1, PW)

    deg_pair = _deg_call(edge_index, zeros_n)
    dinv = lax.rsqrt(deg_pair[0] + deg_pair[1] + 1.0)
    dinv_p = jnp.repeat(dinv, HID).reshape(NP, PW)

    hs1p = _mm1(xp, W1b, dinv_p)
    agg1 = _agg_call(hs1p.reshape(N, HID), edge_index, zeros_nh)
    aggv1 = agg1.reshape(N, PW)
    hs2p = _mid(aggv1, aggv1, hs1p, dinv_p, b1p, W2b)
    agg2 = _agg_call(hs2p.reshape(N, HID), edge_index, zeros_nh)
    aggv2 = agg2.reshape(N, PW)
    hs3p = _mid(aggv2, aggv2, hs2p, dinv_p, b2p, W3b)
    agg3 = _agg_call(hs3p.reshape(N, HID), edge_index, zeros_nh)
    aggv3 = agg3.reshape(N, PW)
    outp = _fin(aggv3, aggv3, hs3p, dinv_p, b3p)
    return outp.reshape(N, C)
